# baseline (speedup 1.0000x reference)
import functools

import jax
import jax.numpy as jnp
from jax import lax
from jax.experimental import pallas as pl
from jax.experimental.pallas import tpu as pltpu
from jax.experimental.pallas import tpu_sc as plsc

_B = 8
_N = 5000
_NP = 5120
_SL = _NP // 4
_NCH = _SL // 16
_C = 80
_D = 100
_NEG = -1e30
_BIG_IDX = 2 ** 30


def _reduce_body(s_ref, best_ref, cls_ref):
    s = s_ref[0]
    m = jnp.max(s, axis=0, keepdims=True)
    ci = lax.broadcasted_iota(jnp.int32, (_C, _N), 0)
    c = jnp.min(jnp.where(s == m, ci, _C), axis=0, keepdims=True)
    best_ref[0] = m
    cls_ref[0] = c


def _sc_nms(best_hbm, cls_hbm, x1_hbm, y1_hbm, x2_hbm, y2_hbm,
            idx_out, sc_out, x1_out, y1_out, x2_out, y2_out, cls_out,
            cnt_out,
            act_v, x1_v, y1_v, x2_v, y2_v, ar_v, cls_v,
            rec_v, grp_v, obuf_v, shared, sem):
    cid = lax.axis_index("c")
    sid = lax.axis_index("s")
    grp = sid // 4
    slc = sid % 4
    b = cid * 4 + grp
    base = slc * _SL

    pltpu.sync_copy(best_hbm.at[b, pl.ds(base, _SL)], act_v)
    pltpu.sync_copy(cls_hbm.at[b, pl.ds(base, _SL)], cls_v.at[pl.ds(0, _SL)])
    pltpu.sync_copy(x1_hbm.at[b, pl.ds(base, _SL)], x1_v.at[pl.ds(0, _SL)])
    pltpu.sync_copy(y1_hbm.at[b, pl.ds(base, _SL)], y1_v.at[pl.ds(0, _SL)])
    pltpu.sync_copy(x2_hbm.at[b, pl.ds(base, _SL)], x2_v.at[pl.ds(0, _SL)])
    pltpu.sync_copy(y2_hbm.at[b, pl.ds(base, _SL)], y2_v.at[pl.ds(0, _SL)])

    def init_chunk(i, carry):
        bmax, bidx = carry
        s = pl.ds(i * 16, 16)
        vx1 = x1_v[s]
        vy1 = y1_v[s]
        vx2 = x2_v[s]
        vy2 = y2_v[s]
        ar_v[s] = (vx2 - vx1) * (vy2 - vy1)
        v = act_v[s]
        v = jnp.where(v > 0.0, v, _NEG)
        act_v[s] = v
        lanes = lax.iota(jnp.int32, 16) + (base + i * 16)
        take = v > bmax
        return jnp.where(take, v, bmax), jnp.where(take, lanes, bidx)

    neg16 = jnp.full((16,), _NEG, jnp.float32)
    big16 = jnp.full((16,), _BIG_IDX, jnp.int32)
    bmax, bidx = lax.fori_loop(0, _NCH, init_chunk, (neg16, big16))

    def lane_argmax(bmax, bidx):
        lane16 = lax.iota(jnp.int32, 16)
        for d in (1, 2, 4, 8):
            perm = jnp.bitwise_xor(lane16, d)
            pm = bmax.at[perm].get(mode="promise_in_bounds")
            pi = bidx.at[perm].get(mode="promise_in_bounds")
            take = (pm > bmax) | ((pm == bmax) & (pi < bidx))
            bmax = jnp.where(take, pm, bmax)
            bidx = jnp.where(take, pi, bidx)
        return bmax[0], bidx[0]

    def step(t, carry):
        bmax, bidx, cnt = carry
        lm, li = lane_argmax(bmax, bidx)
        loc = jnp.minimum(li - base, _SL - 1)
        lane = lax.iota(jnp.int32, 16)
        vx1s = x1_v[pl.ds(loc, 16)][0]
        vy1s = y1_v[pl.ds(loc, 16)][0]
        vx2s = x2_v[pl.ds(loc, 16)][0]
        vy2s = y2_v[pl.ds(loc, 16)][0]
        vcls = cls_v[pl.ds(loc, 16)][0].astype(jnp.float32)
        vars_ = ar_v[pl.ds(loc, 16)][0]
        rec = jnp.where(lane == 0, lm, 0.0)
        rec = jnp.where(lane == 1, li.astype(jnp.float32), rec)
        rec = jnp.where(lane == 2, vx1s, rec)
        rec = jnp.where(lane == 3, vy1s, rec)
        rec = jnp.where(lane == 4, vx2s, rec)
        rec = jnp.where(lane == 5, vy2s, rec)
        rec = jnp.where(lane == 6, vcls, rec)
        rec = jnp.where(lane == 7, vars_, rec)
        rec_v[...] = rec
        pltpu.sync_copy(rec_v, shared.at[16 + sid])
        plsc.subcore_barrier()
        pltpu.sync_copy(shared.at[pl.ds(16 + grp * 4, 4)], grp_v)

        wrec = grp_v[0]
        ws = wrec[0]
        wif = wrec[1]
        for k in range(1, 4):
            rk = grp_v[k]
            ks = rk[0]
            ki = rk[1]
            better = (ks > ws) | ((ks == ws) & (ki < wif))
            bf = jnp.where(better, 1.0, 0.0)
            wrec = wrec * (1.0 - bf) + rk * bf
            ws = jnp.where(better, ks, ws)
            wif = jnp.where(better, ki, wif)
        wx1 = wrec[2]
        wy1 = wrec[3]
        wx2 = wrec[4]
        wy2 = wrec[5]
        wcl = wrec[6]
        war = wrec[7]
        valid = ws > -1e29
        gate = jnp.where(valid, 0.5, 1e30)

        tch = (t // 16) * 16
        tsl = lane == lax.rem(t, 16)

        def put(q, val):
            old = obuf_v[q, pl.ds(tch, 16)]
            obuf_v[q, pl.ds(tch, 16)] = jnp.where(tsl, val, old)

        put(0, jnp.where(valid, ws, 0.0))
        put(1, jnp.where(valid, wif, -1.0))
        put(2, jnp.where(valid, wx1, 0.0))
        put(3, jnp.where(valid, wy1, 0.0))
        put(4, jnp.where(valid, wx2, 0.0))
        put(5, jnp.where(valid, wy2, 0.0))
        put(6, jnp.where(valid, wcl, -1.0))
        cnt = cnt + jnp.where(valid, 1, 0)

        def supp_chunk(i, carry):
            bmax, bidx = carry
            s = pl.ds(i * 16, 16)
            vx1 = x1_v[s]
            vy1 = y1_v[s]
            vx2 = x2_v[s]
            vy2 = y2_v[s]
            var = ar_v[s]
            v = act_v[s]
            iw = jnp.minimum(wx2, vx2) - jnp.maximum(wx1, vx1)
            ih = jnp.minimum(wy2, vy2) - jnp.maximum(wy1, vy1)
            inter = jnp.maximum(iw, 0.0) * jnp.maximum(ih, 0.0)
            union = war + var - inter
            supp = inter > gate * union
            v = jnp.where(supp, _NEG, v)
            act_v[s] = v
            lanes = lax.iota(jnp.int32, 16) + (base + i * 16)
            take = v > bmax
            return jnp.where(take, v, bmax), jnp.where(take, lanes, bidx)

        bmax, bidx = lax.fori_loop(0, _NCH, supp_chunk, (neg16, big16))
        plsc.subcore_barrier()
        return bmax, bidx, cnt

    _, _, cnt = lax.fori_loop(0, _D, step, (bmax, bidx, jnp.int32(0)))

    @pl.when(slc == 0)
    def _():
        pltpu.sync_copy(obuf_v.at[0], sc_out.at[b])
        pltpu.sync_copy(obuf_v.at[1], idx_out.at[b])
        pltpu.sync_copy(obuf_v.at[2], x1_out.at[b])
        pltpu.sync_copy(obuf_v.at[3], y1_out.at[b])
        pltpu.sync_copy(obuf_v.at[4], x2_out.at[b])
        pltpu.sync_copy(obuf_v.at[5], y2_out.at[b])
        pltpu.sync_copy(obuf_v.at[6], cls_out.at[b])
        rec_v[...] = jnp.where(lax.iota(jnp.int32, 16) == 0,
                               cnt.astype(jnp.float32), 0.0)
        pltpu.sync_copy(rec_v, cnt_out.at[b])


@jax.jit
def kernel(scores, boxes):
    scores_t = jnp.swapaxes(scores, 1, 2)
    best, cls = pl.pallas_call(
        _reduce_body,
        grid=(_B,),
        in_specs=[pl.BlockSpec((1, _C, _N), lambda b: (b, 0, 0))],
        out_specs=[pl.BlockSpec((1, 1, _N), lambda b: (b, 0, 0)),
                   pl.BlockSpec((1, 1, _N), lambda b: (b, 0, 0))],
        out_shape=[jax.ShapeDtypeStruct((_B, 1, _N), jnp.float32),
                   jax.ShapeDtypeStruct((_B, 1, _N), jnp.int32)],
    )(scores_t)
    best = jnp.pad(best.reshape(_B, _N), ((0, 0), (0, _NP - _N)),
                   constant_values=-1.0)
    cls = jnp.pad(cls.reshape(_B, _N), ((0, 0), (0, _NP - _N)))
    boxes_t = jnp.transpose(boxes, (2, 0, 1))
    bpad = jnp.pad(boxes_t, ((0, 0), (0, 0), (0, _NP - _N)))
    x1, y1, x2, y2 = bpad[0], bpad[1], bpad[2], bpad[3]

    mesh = plsc.VectorSubcoreMesh(core_axis_name="c", subcore_axis_name="s")
    f32 = jnp.float32
    nms = pl.kernel(
        _sc_nms,
        mesh=mesh,
        out_type=[jax.ShapeDtypeStruct((_B, 128), f32),
                  jax.ShapeDtypeStruct((_B, 128), f32),
                  jax.ShapeDtypeStruct((_B, 128), f32),
                  jax.ShapeDtypeStruct((_B, 128), f32),
                  jax.ShapeDtypeStruct((_B, 128), f32),
                  jax.ShapeDtypeStruct((_B, 128), f32),
                  jax.ShapeDtypeStruct((_B, 128), f32),
                  jax.ShapeDtypeStruct((_B, 16), f32)],
        scratch_types=[
            pltpu.VMEM((_SL,), f32),
            pltpu.VMEM((_SL + 16,), f32),
            pltpu.VMEM((_SL + 16,), f32),
            pltpu.VMEM((_SL + 16,), f32),
            pltpu.VMEM((_SL + 16,), f32),
            pltpu.VMEM((_SL + 16,), f32),
            pltpu.VMEM((_SL + 16,), jnp.int32),
            pltpu.VMEM((16,), f32),
            pltpu.VMEM((4, 16), f32),
            pltpu.VMEM((7, 128), f32),
            pltpu.VMEM_SHARED((32, 16), f32),
            pltpu.SemaphoreType.DMA,
        ],
    )
    oidx, osc, ox1, oy1, ox2, oy2, ocls, ocnt = nms(
        best, cls.astype(jnp.int32), x1, y1, x2, y2)
    boxes_out = jnp.stack([ox1[:, :_D], oy1[:, :_D], ox2[:, :_D],
                           oy2[:, :_D]], axis=-1)
    return (oidx[:, :_D].astype(jnp.int32), osc[:, :_D], boxes_out,
            ocls[:, :_D].astype(jnp.int32), ocnt[:, 0].astype(jnp.int32))
